# Initial kernel scaffold; baseline (speedup 1.0000x reference)
import jax
import jax.numpy as jnp
from jax.experimental import pallas as pl


def kernel(x, edge_index, W1, b1, W2, b2):
    raise NotImplementedError("write your pallas kernel here")


# speedup vs baseline: 25.7643x; 25.7643x over previous
import functools

import jax
import jax.numpy as jnp
from jax import lax
from jax.experimental import pallas as pl
from jax.experimental.pallas import tpu as pltpu
from jax.experimental.pallas import tpu_sc as plsc

N = 10000
D = 128
E = 320000

NC = 2
NS = 16
NW = NC * NS

K = 64
CPW = 160
G = 16
NG = CPW // G
CHUNKS = NW * CPW
EPAD = CHUNKS * K
NPAD = 10240
RPT = NPAD // NS

_mesh = plsc.VectorSubcoreMesh(
    core_axis_name="c", subcore_axis_name="s", num_cores=NC, num_subcores=NS
)


def _wid(cid, sid):
    return sid * NC + cid


@functools.partial(
    pl.kernel,
    out_type=jax.ShapeDtypeStruct((NC, NPAD), jnp.float32),
    mesh=_mesh,
    scratch_types=[
        pltpu.VMEM((CPW, K), jnp.int32),
        pltpu.VMEM((K,), jnp.float32),
        pltpu.VMEM_SHARED((NPAD,), jnp.float32),
    ],
)
def _deg_kernel(dst_hbm, zeros1_hbm, degp_hbm, dst_v, ones_v, degacc):
    cid = lax.axis_index("c")
    sid = lax.axis_index("s")
    wid = _wid(cid, sid)
    for i in range(K // 16):
        ones_v[pl.ds(i * 16, 16)] = jnp.ones((16,), jnp.float32)
    pltpu.sync_copy(zeros1_hbm.at[pl.ds(sid * RPT, RPT)],
                    degacc.at[pl.ds(sid * RPT, RPT)])
    pltpu.sync_copy(dst_hbm.at[pl.ds(wid * CPW, CPW)], dst_v)
    plsc.subcore_barrier()
    def body(j, _):
        pltpu.sync_copy(ones_v, degacc.at[dst_v.at[j]], add=True)
        return ()
    lax.fori_loop(0, CPW, body, ())
    plsc.subcore_barrier()
    pltpu.sync_copy(degacc.at[pl.ds(sid * RPT, RPT)],
                    degp_hbm.at[cid, pl.ds(sid * RPT, RPT)])


@functools.partial(
    pl.kernel,
    out_type=jax.ShapeDtypeStruct((NC, NPAD, D), jnp.float32),
    mesh=_mesh,
    scratch_types=[
        pltpu.VMEM((G, K), jnp.int32),
        pltpu.VMEM((G, K), jnp.int32),
        pltpu.VMEM((2, K, D), jnp.float32),
        pltpu.VMEM_SHARED((NPAD, D), jnp.float32),
        pltpu.SemaphoreType.DMA((2,)),
    ],
)
def _acc_kernel(hs_hbm, src_hbm, dst_hbm, zeros2_hbm, accp_hbm,
                src_v, dst_v, rows_v, acc, sems):
    cid = lax.axis_index("c")
    sid = lax.axis_index("s")
    wid = _wid(cid, sid)
    pltpu.sync_copy(zeros2_hbm.at[pl.ds(sid * RPT, RPT)],
                    acc.at[pl.ds(sid * RPT, RPT)])
    plsc.subcore_barrier()

    def group(g, _):
        base = wid * CPW + g * G
        pltpu.sync_copy(src_hbm.at[pl.ds(base, G)], src_v)
        pltpu.sync_copy(dst_hbm.at[pl.ds(base, G)], dst_v)
        pltpu.async_copy(hs_hbm.at[src_v.at[0]], rows_v.at[0], sems.at[0])

        def body(j, _):
            b = lax.rem(j, 2)
            nb = lax.rem(j + 1, 2)
            pltpu.async_copy(hs_hbm.at[src_v.at[j + 1]], rows_v.at[nb],
                             sems.at[nb])
            pltpu.make_async_copy(hs_hbm.at[src_v.at[j]], rows_v.at[b],
                                  sems.at[b]).wait()
            pltpu.sync_copy(rows_v.at[b], acc.at[dst_v.at[j]], add=True)
            return ()

        lax.fori_loop(0, G - 1, body, ())
        b = (G - 1) % 2
        pltpu.make_async_copy(hs_hbm.at[src_v.at[G - 1]], rows_v.at[b],
                              sems.at[b]).wait()
        pltpu.sync_copy(rows_v.at[b], acc.at[dst_v.at[G - 1]], add=True)
        return ()

    lax.fori_loop(0, NG, group, ())
    plsc.subcore_barrier()
    pltpu.sync_copy(acc.at[pl.ds(sid * RPT, RPT)],
                    accp_hbm.at[cid, pl.ds(sid * RPT, RPT)])


def _dinv_from(degt_ref):
    deg = 1.0 + jnp.sum(degt_ref[...], axis=1, keepdims=True)
    return lax.rsqrt(deg)


def _tc1_body(x_ref, w_ref, degt_ref, hs_ref):
    dinv = _dinv_from(degt_ref)
    hs_ref[...] = jnp.dot(x_ref[...], w_ref[...],
                          preferred_element_type=jnp.float32) * dinv


def _tc2_body(accp_ref, hs1_ref, degt_ref, b_ref, w_ref, hs2_ref):
    dinv = _dinv_from(degt_ref)
    t = dinv * (accp_ref[0] + accp_ref[1] + hs1_ref[...]) + b_ref[...]
    t = jnp.maximum(t, 0.0)
    hs2_ref[...] = jnp.dot(t, w_ref[...],
                           preferred_element_type=jnp.float32) * dinv


def _tc3_body(accp_ref, hs2_ref, degt_ref, b_ref, out_ref):
    dinv = _dinv_from(degt_ref)
    t = dinv * (accp_ref[0] + accp_ref[1] + hs2_ref[...]) + b_ref[...]
    out_ref[...] = jnp.maximum(t, 0.0)


_tc1 = pl.pallas_call(
    _tc1_body, out_shape=jax.ShapeDtypeStruct((NPAD, D), jnp.float32))
_tc2 = pl.pallas_call(
    _tc2_body, out_shape=jax.ShapeDtypeStruct((NPAD, D), jnp.float32))
_tc3 = pl.pallas_call(
    _tc3_body, out_shape=jax.ShapeDtypeStruct((NPAD, D), jnp.float32))


def kernel(x, edge_index, W1, b1, W2, b2):
    src = edge_index[0]
    dst = edge_index[1]
    pad_idx = (N + jnp.arange(EPAD - E, dtype=jnp.int32) % (NPAD - N)).astype(
        jnp.int32)
    src_p = jnp.concatenate([src, pad_idx]).reshape(CHUNKS, K)
    dst_p = jnp.concatenate([dst, pad_idx]).reshape(CHUNKS, K)
    x_pad = jnp.pad(x, ((0, NPAD - N), (0, 0)))
    zeros1 = jnp.zeros((NPAD,), jnp.float32)
    zeros2 = jnp.zeros((NPAD, D), jnp.float32)

    degp = _deg_kernel(dst_p, zeros1)
    degt = degp.T
    hs1 = _tc1(x_pad, W1, degt)
    accp1 = _acc_kernel(hs1, src_p, dst_p, zeros2)
    hs2 = _tc2(accp1, hs1, degt, b1.reshape(1, D), W2)
    accp2 = _acc_kernel(hs2, src_p, dst_p, zeros2)
    out = _tc3(accp2, hs2, degt, b2.reshape(1, D))
    return out[:N]


# speedup vs baseline: 28.4221x; 1.1032x over previous
import functools

import jax
import jax.numpy as jnp
from jax import lax
from jax.experimental import pallas as pl
from jax.experimental.pallas import tpu as pltpu
from jax.experimental.pallas import tpu_sc as plsc

N = 10000
D = 128
E = 320000

NC = 2
NS = 16
NW = NC * NS

K = 64
CPW = 160
G = 32
NG = CPW // G
CHUNKS = NW * CPW
EPAD = CHUNKS * K
NPAD = 10240
RPT = NPAD // NS

_mesh = plsc.VectorSubcoreMesh(
    core_axis_name="c", subcore_axis_name="s", num_cores=NC, num_subcores=NS
)


def _wid(cid, sid):
    return sid * NC + cid


@functools.partial(
    pl.kernel,
    out_type=jax.ShapeDtypeStruct((NC, NPAD), jnp.float32),
    mesh=_mesh,
    scratch_types=[
        pltpu.VMEM((CPW, K), jnp.int32),
        pltpu.VMEM((K,), jnp.float32),
        pltpu.VMEM_SHARED((NPAD,), jnp.float32),
    ],
)
def _deg_kernel(dst_hbm, zeros1_hbm, degp_hbm, dst_v, ones_v, degacc):
    cid = lax.axis_index("c")
    sid = lax.axis_index("s")
    wid = _wid(cid, sid)
    for i in range(K // 16):
        ones_v[pl.ds(i * 16, 16)] = jnp.ones((16,), jnp.float32)
    pltpu.sync_copy(zeros1_hbm.at[pl.ds(sid * RPT, RPT)],
                    degacc.at[pl.ds(sid * RPT, RPT)])
    pltpu.sync_copy(dst_hbm.at[pl.ds(wid * CPW, CPW)], dst_v)
    plsc.subcore_barrier()

    def body(j, _):
        pltpu.sync_copy(ones_v, degacc.at[dst_v.at[j]], add=True)
        return ()

    lax.fori_loop(0, CPW, body, ())
    plsc.subcore_barrier()
    pltpu.sync_copy(degacc.at[pl.ds(sid * RPT, RPT)],
                    degp_hbm.at[cid, pl.ds(sid * RPT, RPT)])


@functools.partial(
    pl.kernel,
    out_type=jax.ShapeDtypeStruct((NC, NPAD, D), jnp.float32),
    mesh=_mesh,
    scratch_types=[
        pltpu.VMEM((2, G, K), jnp.int32),
        pltpu.VMEM((2, G, K), jnp.int32),
        pltpu.VMEM((2, K, D), jnp.float32),
        pltpu.VMEM_SHARED((NPAD, D), jnp.float32),
        pltpu.SemaphoreType.DMA((2,)),
        pltpu.SemaphoreType.DMA((2,)),
    ],
)
def _acc_kernel(hs_hbm, src_hbm, dst_hbm, zeros2_hbm, accp_hbm,
                src_v, dst_v, rows_v, acc, isems, gsems):
    cid = lax.axis_index("c")
    sid = lax.axis_index("s")
    wid = _wid(cid, sid)

    @pl.when(cid == 0)
    def _():
        pltpu.sync_copy(hs_hbm.at[pl.ds(sid * RPT, RPT)],
                        acc.at[pl.ds(sid * RPT, RPT)])

    @pl.when(cid != 0)
    def _():
        pltpu.sync_copy(zeros2_hbm.at[pl.ds(sid * RPT, RPT)],
                        acc.at[pl.ds(sid * RPT, RPT)])

    base0 = wid * CPW
    pltpu.async_copy(src_hbm.at[pl.ds(base0, G)], src_v.at[0], isems.at[0])
    pltpu.async_copy(dst_hbm.at[pl.ds(base0, G)], dst_v.at[0], isems.at[0])
    pltpu.async_copy(src_hbm.at[pl.ds(base0 + G, G)], src_v.at[1],
                     isems.at[1])
    pltpu.async_copy(dst_hbm.at[pl.ds(base0 + G, G)], dst_v.at[1],
                     isems.at[1])
    plsc.subcore_barrier()

    def group(g, _):
        gb = lax.rem(g, 2)
        base = wid * CPW + g * G
        pltpu.make_async_copy(src_hbm.at[pl.ds(base, G)], src_v.at[gb],
                              isems.at[gb]).wait()
        pltpu.make_async_copy(dst_hbm.at[pl.ds(base, G)], dst_v.at[gb],
                              isems.at[gb]).wait()

        def gather(j, b):
            pltpu.async_copy(hs_hbm.at[src_v.at[gb, j]], rows_v.at[b],
                             gsems.at[b])

        def wait_scatter(j, b):
            pltpu.make_async_copy(hs_hbm.at[src_v.at[gb, j]], rows_v.at[b],
                                  gsems.at[b]).wait()
            pltpu.sync_copy(rows_v.at[b], acc.at[dst_v.at[gb, j]], add=True)

        gather(0, 0)

        def pair(p, _):
            j0 = 2 * p
            gather(j0 + 1, 1)
            wait_scatter(j0, 0)
            gather(j0 + 2, 0)
            wait_scatter(j0 + 1, 1)
            return ()

        lax.fori_loop(0, G // 2 - 1, pair, ())
        gather(G - 1, 1)
        wait_scatter(G - 2, 0)
        wait_scatter(G - 1, 1)

        @pl.when(g + 2 < NG)
        def _():
            nbase = wid * CPW + (g + 2) * G
            pltpu.async_copy(src_hbm.at[pl.ds(nbase, G)], src_v.at[gb],
                             isems.at[gb])
            pltpu.async_copy(dst_hbm.at[pl.ds(nbase, G)], dst_v.at[gb],
                             isems.at[gb])
        return ()

    lax.fori_loop(0, NG, group, ())
    plsc.subcore_barrier()
    pltpu.sync_copy(acc.at[pl.ds(sid * RPT, RPT)],
                    accp_hbm.at[cid, pl.ds(sid * RPT, RPT)])


def _dinv_from(degt_ref):
    deg = 1.0 + jnp.sum(degt_ref[...], axis=1, keepdims=True)
    return lax.rsqrt(deg)


def _tc_mm_body(x_ref, w_ref, h_ref):
    h_ref[:N, :] = jnp.dot(x_ref[...], w_ref[...],
                           preferred_element_type=jnp.float32)
    h_ref[N:, :] = jnp.zeros((NPAD - N, D), jnp.float32)


def _tc_scale_body(h_ref, degt_ref, hs_ref):
    hs_ref[...] = h_ref[...] * _dinv_from(degt_ref)


def _tc2_body(accp_ref, degt_ref, b_ref, w_ref, hs2_ref):
    dinv = _dinv_from(degt_ref)
    t = dinv * (accp_ref[0] + accp_ref[1]) + b_ref[...]
    t = jnp.maximum(t, 0.0)
    hs2_ref[...] = jnp.dot(t, w_ref[...],
                           preferred_element_type=jnp.float32) * dinv


def _tc3_body(accp_ref, degt_ref, b_ref, out_ref):
    dinv = _dinv_from(degt_ref)
    t = dinv[:N, :] * (accp_ref[0, :N, :] + accp_ref[1, :N, :]) + b_ref[...]
    out_ref[...] = jnp.maximum(t, 0.0)


_tc_mm = pl.pallas_call(
    _tc_mm_body, out_shape=jax.ShapeDtypeStruct((NPAD, D), jnp.float32))
_tc_scale = pl.pallas_call(
    _tc_scale_body, out_shape=jax.ShapeDtypeStruct((NPAD, D), jnp.float32))
_tc2 = pl.pallas_call(
    _tc2_body, out_shape=jax.ShapeDtypeStruct((NPAD, D), jnp.float32))
_tc3 = pl.pallas_call(
    _tc3_body, out_shape=jax.ShapeDtypeStruct((N, D), jnp.float32))


def kernel(x, edge_index, W1, b1, W2, b2):
    src = edge_index[0]
    dst = edge_index[1]
    pad_idx = (N + jnp.arange(EPAD - E, dtype=jnp.int32) % (NPAD - N)).astype(
        jnp.int32)
    src_p = jnp.concatenate([src, pad_idx]).reshape(CHUNKS, K)
    dst_p = jnp.concatenate([dst, pad_idx]).reshape(CHUNKS, K)
    zeros1 = jnp.zeros((NPAD,), jnp.float32)
    zeros2 = jnp.zeros((NPAD, D), jnp.float32)

    h1 = _tc_mm(x, W1)
    degp = _deg_kernel(dst_p, zeros1)
    degt = degp.T
    hs1 = _tc_scale(h1, degt)
    accp1 = _acc_kernel(hs1, src_p, dst_p, zeros2)
    hs2 = _tc2(accp1, degt, b1.reshape(1, D), W2)
    accp2 = _acc_kernel(hs2, src_p, dst_p, zeros2)
    return _tc3(accp2, degt, b2.reshape(1, D))


# speedup vs baseline: 28.9240x; 1.0177x over previous
import functools

import jax
import jax.numpy as jnp
from jax import lax
from jax.experimental import pallas as pl
from jax.experimental.pallas import tpu as pltpu
from jax.experimental.pallas import tpu_sc as plsc

N = 10000
D = 128
E = 320000

NC = 2
NS = 16
NW = NC * NS

K = 64
CPW = 160
G = 32
NG = CPW // G
CHUNKS = NW * CPW
EPAD = CHUNKS * K
NPAD = 10240
RPT = NPAD // NS

_mesh = plsc.VectorSubcoreMesh(
    core_axis_name="c", subcore_axis_name="s", num_cores=NC, num_subcores=NS
)


def _wid(cid, sid):
    return sid * NC + cid


KD = 128
CPWD = EPAD // (NW * KD)


@functools.partial(
    pl.kernel,
    out_type=jax.ShapeDtypeStruct((NC, NPAD), jnp.float32),
    mesh=_mesh,
    scratch_types=[
        pltpu.VMEM((CPWD, KD), jnp.int32),
        pltpu.VMEM((KD,), jnp.float32),
        pltpu.VMEM_SHARED((NPAD,), jnp.float32),
    ],
)
def _deg_kernel(dst_hbm, zeros1_hbm, degp_hbm, dst_v, ones_v, degacc):
    cid = lax.axis_index("c")
    sid = lax.axis_index("s")
    wid = _wid(cid, sid)
    for i in range(KD // 16):
        ones_v[pl.ds(i * 16, 16)] = jnp.ones((16,), jnp.float32)
    pltpu.sync_copy(zeros1_hbm.at[pl.ds(sid * RPT, RPT)],
                    degacc.at[pl.ds(sid * RPT, RPT)])
    pltpu.sync_copy(dst_hbm.at[pl.ds(wid * CPWD, CPWD)], dst_v)
    plsc.subcore_barrier()

    def body(j, _):
        pltpu.sync_copy(ones_v, degacc.at[dst_v.at[j]], add=True)
        return ()

    lax.fori_loop(0, CPWD, body, ())
    plsc.subcore_barrier()
    pltpu.sync_copy(degacc.at[pl.ds(sid * RPT, RPT)],
                    degp_hbm.at[cid, pl.ds(sid * RPT, RPT)])


@functools.partial(
    pl.kernel,
    out_type=jax.ShapeDtypeStruct((NC, NPAD, D), jnp.float32),
    mesh=_mesh,
    scratch_types=[
        pltpu.VMEM((2, G, K), jnp.int32),
        pltpu.VMEM((2, G, K), jnp.int32),
        pltpu.VMEM((2, K, D), jnp.float32),
        pltpu.VMEM_SHARED((NPAD, D), jnp.float32),
        pltpu.SemaphoreType.DMA((2,)),
        pltpu.SemaphoreType.DMA((2,)),
    ],
)
def _acc_kernel(hs_hbm, src_hbm, dst_hbm, zeros2_hbm, accp_hbm,
                src_v, dst_v, rows_v, acc, isems, gsems):
    cid = lax.axis_index("c")
    sid = lax.axis_index("s")
    wid = _wid(cid, sid)

    @pl.when(cid == 0)
    def _():
        pltpu.sync_copy(hs_hbm.at[pl.ds(sid * RPT, RPT)],
                        acc.at[pl.ds(sid * RPT, RPT)])

    @pl.when(cid != 0)
    def _():
        pltpu.sync_copy(zeros2_hbm.at[pl.ds(sid * RPT, RPT)],
                        acc.at[pl.ds(sid * RPT, RPT)])

    base0 = wid * CPW
    pltpu.async_copy(src_hbm.at[pl.ds(base0, G)], src_v.at[0], isems.at[0])
    pltpu.async_copy(dst_hbm.at[pl.ds(base0, G)], dst_v.at[0], isems.at[0])
    pltpu.async_copy(src_hbm.at[pl.ds(base0 + G, G)], src_v.at[1],
                     isems.at[1])
    pltpu.async_copy(dst_hbm.at[pl.ds(base0 + G, G)], dst_v.at[1],
                     isems.at[1])
    plsc.subcore_barrier()

    def group(g, _):
        gb = lax.rem(g, 2)
        base = wid * CPW + g * G
        pltpu.make_async_copy(src_hbm.at[pl.ds(base, G)], src_v.at[gb],
                              isems.at[gb]).wait()
        pltpu.make_async_copy(dst_hbm.at[pl.ds(base, G)], dst_v.at[gb],
                              isems.at[gb]).wait()

        def gather(j, b):
            pltpu.async_copy(hs_hbm.at[src_v.at[gb, j]], rows_v.at[b],
                             gsems.at[b])

        def wait_scatter(j, b):
            pltpu.make_async_copy(hs_hbm.at[src_v.at[gb, j]], rows_v.at[b],
                                  gsems.at[b]).wait()
            pltpu.sync_copy(rows_v.at[b], acc.at[dst_v.at[gb, j]], add=True)

        gather(0, 0)

        def pair(p, _):
            j0 = 2 * p
            gather(j0 + 1, 1)
            wait_scatter(j0, 0)
            gather(j0 + 2, 0)
            wait_scatter(j0 + 1, 1)
            return ()

        lax.fori_loop(0, G // 2 - 1, pair, ())
        gather(G - 1, 1)
        wait_scatter(G - 2, 0)
        wait_scatter(G - 1, 1)

        @pl.when(g + 2 < NG)
        def _():
            nbase = wid * CPW + (g + 2) * G
            pltpu.async_copy(src_hbm.at[pl.ds(nbase, G)], src_v.at[gb],
                             isems.at[gb])
            pltpu.async_copy(dst_hbm.at[pl.ds(nbase, G)], dst_v.at[gb],
                             isems.at[gb])
        return ()

    lax.fori_loop(0, NG, group, ())
    plsc.subcore_barrier()
    pltpu.sync_copy(acc.at[pl.ds(sid * RPT, RPT)],
                    accp_hbm.at[cid, pl.ds(sid * RPT, RPT)])


def _dinv_from(degt_ref):
    deg = 1.0 + jnp.sum(degt_ref[...], axis=1, keepdims=True)
    return lax.rsqrt(deg)


def _tc_mm_body(x_ref, w_ref, h_ref):
    h_ref[:N, :] = jnp.dot(x_ref[...], w_ref[...],
                           preferred_element_type=jnp.float32)
    h_ref[N:, :] = jnp.zeros((NPAD - N, D), jnp.float32)


def _tc_scale_body(h_ref, degt_ref, hs_ref):
    hs_ref[...] = h_ref[...] * _dinv_from(degt_ref)


def _tc2_body(accp_ref, degt_ref, b_ref, w_ref, hs2_ref):
    dinv = _dinv_from(degt_ref)
    t = dinv * (accp_ref[0] + accp_ref[1]) + b_ref[...]
    t = jnp.maximum(t, 0.0)
    hs2_ref[...] = jnp.dot(t, w_ref[...],
                           preferred_element_type=jnp.float32) * dinv


def _tc3_body(accp_ref, degt_ref, b_ref, out_ref):
    dinv = _dinv_from(degt_ref)
    t = dinv[:N, :] * (accp_ref[0, :N, :] + accp_ref[1, :N, :]) + b_ref[...]
    out_ref[...] = jnp.maximum(t, 0.0)


_tc_mm = pl.pallas_call(
    _tc_mm_body, out_shape=jax.ShapeDtypeStruct((NPAD, D), jnp.float32))
_tc_scale = pl.pallas_call(
    _tc_scale_body, out_shape=jax.ShapeDtypeStruct((NPAD, D), jnp.float32))
_tc2 = pl.pallas_call(
    _tc2_body, out_shape=jax.ShapeDtypeStruct((NPAD, D), jnp.float32))
_tc3 = pl.pallas_call(
    _tc3_body, out_shape=jax.ShapeDtypeStruct((N, D), jnp.float32))


def kernel(x, edge_index, W1, b1, W2, b2):
    src = edge_index[0]
    dst = edge_index[1]
    pad_idx = (N + jnp.arange(EPAD - E, dtype=jnp.int32) % (NPAD - N)).astype(
        jnp.int32)
    src_p = jnp.concatenate([src, pad_idx]).reshape(CHUNKS, K)
    dst_pad = jnp.concatenate([dst, pad_idx])
    dst_p = dst_pad.reshape(CHUNKS, K)
    dst_pd = dst_pad.reshape(EPAD // KD, KD)
    zeros1 = jnp.zeros((NPAD,), jnp.float32)
    zeros2 = jnp.zeros((NPAD, D), jnp.float32)

    h1 = _tc_mm(x, W1)
    degp = _deg_kernel(dst_pd, zeros1)
    degt = degp.T
    hs1 = _tc_scale(h1, degt)
    accp1 = _acc_kernel(hs1, src_p, dst_p, zeros2)
    hs2 = _tc2(accp1, degt, b1.reshape(1, D), W2)
    accp2 = _acc_kernel(hs2, src_p, dst_p, zeros2)
    return _tc3(accp2, degt, b2.reshape(1, D))
